# initial kernel scaffold (unmeasured)
import jax
import jax.numpy as jnp
from jax import lax
from jax.experimental import pallas as pl
from jax.experimental.pallas import tpu as pltpu

N_DEV = 32
BLK = 64


def kernel(x, w_mat):
    m_glob, k_per = x.shape
    k_glob, n = w_mat.shape
    m_per = m_glob // N_DEV

    def body(x_ref, w_ref, out_ref, asm_ref, send_sems, recv_sems):
        me = lax.axis_index("i")

        asm_ref[:, pl.ds(me * BLK, BLK)] = x_ref[pl.ds(me * m_per, m_per), :]

        for j in range(N_DEV):
            @pl.when(j != me)
            def _():
                rdma = pltpu.make_async_remote_copy(
                    src_ref=x_ref.at[pl.ds(j * m_per, m_per), :],
                    dst_ref=asm_ref.at[:, pl.ds(me * BLK, BLK)],
                    send_sem=send_sems.at[j],
                    recv_sem=recv_sems.at[me],
                    device_id=(j,),
                    device_id_type=pl.DeviceIdType.MESH,
                )
                rdma.start()

        for j in range(N_DEV):
            @pl.when(j != me)
            def _():
                recv = pltpu.make_async_remote_copy(
                    src_ref=x_ref.at[pl.ds(j * m_per, m_per), :],
                    dst_ref=asm_ref.at[:, pl.ds(j * BLK, BLK)],
                    send_sem=send_sems.at[j],
                    recv_sem=recv_sems.at[j],
                    device_id=(j,),
                    device_id_type=pl.DeviceIdType.MESH,
                )
                recv.wait_recv()

        for j in range(N_DEV):
            @pl.when(j != me)
            def _():
                send = pltpu.make_async_remote_copy(
                    src_ref=x_ref.at[pl.ds(j * m_per, m_per), :],
                    dst_ref=asm_ref.at[:, pl.ds(j * BLK, BLK)],
                    send_sem=send_sems.at[j],
                    recv_sem=recv_sems.at[j],
                    device_id=(j,),
                    device_id_type=pl.DeviceIdType.MESH,
                )
                send.wait_send()

        out_ref[:, :] = jnp.dot(
            asm_ref[:, :], w_ref[:, :], preferred_element_type=jnp.float32
        )

    return pl.pallas_call(
        body,
        out_shape=jax.ShapeDtypeStruct((m_per, n), jnp.float32),
        in_specs=[
            pl.BlockSpec(memory_space=pltpu.VMEM),
            pl.BlockSpec(memory_space=pltpu.VMEM),
        ],
        out_specs=pl.BlockSpec(memory_space=pltpu.VMEM),
        scratch_shapes=[
            pltpu.VMEM((m_per, k_glob), jnp.float32),
            pltpu.SemaphoreType.DMA((N_DEV,)),
            pltpu.SemaphoreType.DMA((N_DEV,)),
        ],
        compiler_params=pltpu.CompilerParams(collective_id=0),
    )(x, w_mat)


# baseline (device time: 34035 ns/iter reference)
import jax
import jax.numpy as jnp
from jax import lax
from jax.experimental import pallas as pl
from jax.experimental.pallas import tpu as pltpu

N_DEV = 32
BLK = 64


def kernel(x, w_mat):
    m_glob, k_per = x.shape
    k_glob, n = w_mat.shape
    m_per = m_glob // N_DEV

    def body(x_ref, w_ref, out_ref, asm_ref, send_sems, recv_sems):
        me = lax.axis_index("i")

        asm_ref[me] = x_ref[pl.ds(me * m_per, m_per), :]

        for j in range(N_DEV):
            @pl.when(j != me)
            def _():
                rdma = pltpu.make_async_remote_copy(
                    src_ref=x_ref.at[pl.ds(j * m_per, m_per), :],
                    dst_ref=asm_ref.at[me],
                    send_sem=send_sems.at[j],
                    recv_sem=recv_sems.at[me],
                    device_id=(j,),
                    device_id_type=pl.DeviceIdType.MESH,
                )
                rdma.start()

        for j in range(N_DEV):
            @pl.when(j != me)
            def _():
                recv = pltpu.make_async_remote_copy(
                    src_ref=x_ref.at[pl.ds(j * m_per, m_per), :],
                    dst_ref=asm_ref.at[j],
                    send_sem=send_sems.at[j],
                    recv_sem=recv_sems.at[j],
                    device_id=(j,),
                    device_id_type=pl.DeviceIdType.MESH,
                )
                recv.wait_recv()

        for j in range(N_DEV):
            @pl.when(j != me)
            def _():
                send = pltpu.make_async_remote_copy(
                    src_ref=x_ref.at[pl.ds(j * m_per, m_per), :],
                    dst_ref=asm_ref.at[j],
                    send_sem=send_sems.at[j],
                    recv_sem=recv_sems.at[j],
                    device_id=(j,),
                    device_id_type=pl.DeviceIdType.MESH,
                )
                send.wait_send()

        a = jnp.transpose(asm_ref[...], (1, 0, 2)).reshape(m_per, k_glob)
        out_ref[:, :] = jnp.dot(
            a, w_ref[:, :], preferred_element_type=jnp.float32
        )

    return pl.pallas_call(
        body,
        out_shape=jax.ShapeDtypeStruct((m_per, n), jnp.float32),
        in_specs=[
            pl.BlockSpec(memory_space=pltpu.VMEM),
            pl.BlockSpec(memory_space=pltpu.VMEM),
        ],
        out_specs=pl.BlockSpec(memory_space=pltpu.VMEM),
        scratch_shapes=[
            pltpu.VMEM((N_DEV, m_per, BLK), jnp.float32),
            pltpu.SemaphoreType.DMA((N_DEV,)),
            pltpu.SemaphoreType.DMA((N_DEV,)),
        ],
    )(x, w_mat)


# device time: 27905 ns/iter; 1.2197x vs baseline; 1.2197x over previous
import jax
import jax.numpy as jnp
from jax import lax
from jax.experimental import pallas as pl
from jax.experimental.pallas import tpu as pltpu

N_DEV = 32
BLK = 64
HALF = 32
W_CHUNKS = 4


def kernel(x, w_mat):
    m_glob, k_per = x.shape
    k_glob, n = w_mat.shape
    m_per = m_glob // N_DEV

    x2 = (
        x.reshape(N_DEV, 2, HALF, BLK)
        .transpose(0, 2, 1, 3)
        .reshape(N_DEV * HALF, 2 * BLK)
    )

    def body(x2_ref, w_hbm, out_ref, w_vmem, asm_ref, send_sems, recv_sems,
             w_sems):
        me = lax.axis_index("i")

        rows = k_glob // W_CHUNKS
        for c in range(W_CHUNKS):
            pltpu.make_async_copy(
                w_hbm.at[pl.ds(c * rows, rows), :],
                w_vmem.at[pl.ds(c * rows, rows), :],
                w_sems.at[c],
            ).start()

        bar = pltpu.get_barrier_semaphore()
        for j in range(N_DEV):
            @pl.when(j != me)
            def _():
                pl.semaphore_signal(
                    bar, inc=1, device_id=(j,),
                    device_id_type=pl.DeviceIdType.MESH,
                )
        pl.semaphore_wait(bar, N_DEV - 1)

        asm_ref[me] = x2_ref[pl.ds(me * HALF, HALF), :]

        for d in range(1, N_DEV):
            j = lax.rem(me + d, N_DEV)
            pltpu.make_async_remote_copy(
                src_ref=x2_ref.at[pl.ds(j * HALF, HALF), :],
                dst_ref=asm_ref.at[me],
                send_sem=send_sems.at[d],
                recv_sem=recv_sems.at[me],
                device_id=(j,),
                device_id_type=pl.DeviceIdType.MESH,
            ).start()

        for d in range(1, N_DEV):
            s = lax.rem(me - d + N_DEV, N_DEV)
            pltpu.make_async_remote_copy(
                src_ref=x2_ref.at[pl.ds(s * HALF, HALF), :],
                dst_ref=asm_ref.at[s],
                send_sem=send_sems.at[d],
                recv_sem=recv_sems.at[s],
                device_id=(s,),
                device_id_type=pl.DeviceIdType.MESH,
            ).wait_recv()

        for d in range(1, N_DEV):
            s = lax.rem(me - d + N_DEV, N_DEV)
            pltpu.make_async_remote_copy(
                src_ref=x2_ref.at[pl.ds(s * HALF, HALF), :],
                dst_ref=asm_ref.at[s],
                send_sem=send_sems.at[d],
                recv_sem=recv_sems.at[s],
                device_id=(s,),
                device_id_type=pl.DeviceIdType.MESH,
            ).wait_send()

        for c in range(W_CHUNKS):
            pltpu.make_async_copy(
                w_hbm.at[pl.ds(c * rows, rows), :],
                w_vmem.at[pl.ds(c * rows, rows), :],
                w_sems.at[c],
            ).wait()

        a3 = asm_ref[...]
        top = jnp.transpose(a3[:, :, 0:BLK], (1, 0, 2)).reshape(HALF, k_glob)
        bot = jnp.transpose(a3[:, :, BLK:2 * BLK], (1, 0, 2)).reshape(
            HALF, k_glob
        )
        a = jnp.concatenate([top, bot], axis=0)
        out_ref[:, :] = jnp.dot(
            a, w_vmem[:, :], preferred_element_type=jnp.float32
        )

    return pl.pallas_call(
        body,
        out_shape=jax.ShapeDtypeStruct((m_per, n), jnp.float32),
        in_specs=[
            pl.BlockSpec(memory_space=pltpu.VMEM),
            pl.BlockSpec(memory_space=pltpu.MemorySpace.HBM),
        ],
        out_specs=pl.BlockSpec(memory_space=pltpu.VMEM),
        scratch_shapes=[
            pltpu.VMEM((k_glob, n), jnp.float32),
            pltpu.VMEM((N_DEV, HALF, 2 * BLK), jnp.float32),
            pltpu.SemaphoreType.DMA((N_DEV,)),
            pltpu.SemaphoreType.DMA((N_DEV,)),
            pltpu.SemaphoreType.DMA((W_CHUNKS,)),
        ],
        compiler_params=pltpu.CompilerParams(collective_id=0),
    )(x2, w_mat)


# device time: 26405 ns/iter; 1.2890x vs baseline; 1.0568x over previous
import jax
import jax.numpy as jnp
from jax import lax
from jax.experimental import pallas as pl
from jax.experimental.pallas import tpu as pltpu

N_DEV = 32
BLK = 64
HALF = 32
GROUPS = 4
PER_G = N_DEV // GROUPS


def kernel(x, w_mat):
    m_glob, k_per = x.shape
    k_glob, n = w_mat.shape
    m_per = m_glob // N_DEV

    x2 = (
        x.reshape(N_DEV, 2, HALF, BLK)
        .transpose(0, 2, 1, 3)
        .reshape(N_DEV * HALF, 2 * BLK)
    )

    def body(x2_ref, w_ref, out_ref, asm_ref, send_sems, recv_sems,
             ready_sems):
        me = lax.axis_index("i")

        bar = pltpu.get_barrier_semaphore()
        pl.semaphore_signal(bar, inc=1, device_id=(me,),
                            device_id_type=pl.DeviceIdType.MESH)
        pl.semaphore_wait(bar, 1)

        for j in range(N_DEV):
            @pl.when(j != me)
            def _():
                pl.semaphore_signal(
                    ready_sems.at[me], inc=1, device_id=(j,),
                    device_id_type=pl.DeviceIdType.MESH,
                )

        asm_ref[me] = x2_ref[pl.ds(me * HALF, HALF), :]

        for d in range(1, N_DEV):
            t = lax.rem(me + d, N_DEV)
            pl.semaphore_wait(ready_sems.at[t], 1)
            pltpu.make_async_remote_copy(
                src_ref=x2_ref.at[pl.ds(t * HALF, HALF), :],
                dst_ref=asm_ref.at[me],
                send_sem=send_sems.at[d],
                recv_sem=recv_sems.at[me],
                device_id=(t,),
                device_id_type=pl.DeviceIdType.MESH,
            ).start()

        for g in range(GROUPS):
            for s in range(g * PER_G, (g + 1) * PER_G):
                @pl.when(s != me)
                def _():
                    pltpu.make_async_remote_copy(
                        src_ref=x2_ref.at[pl.ds(s * HALF, HALF), :],
                        dst_ref=asm_ref.at[s],
                        send_sem=send_sems.at[1],
                        recv_sem=recv_sems.at[s],
                        device_id=(s,),
                        device_id_type=pl.DeviceIdType.MESH,
                    ).wait_recv()
            a3 = asm_ref[pl.ds(g * PER_G, PER_G)]
            top = jnp.transpose(a3[:, :, 0:BLK], (1, 0, 2)).reshape(
                HALF, PER_G * BLK
            )
            bot = jnp.transpose(a3[:, :, BLK:2 * BLK], (1, 0, 2)).reshape(
                HALF, PER_G * BLK
            )
            a_g = jnp.concatenate([top, bot], axis=0)
            part = jnp.dot(
                a_g,
                w_ref[pl.ds(g * PER_G * BLK, PER_G * BLK), :],
                preferred_element_type=jnp.float32,
            )
            if g == 0:
                out_ref[:, :] = part
            else:
                out_ref[:, :] += part

        for d in range(1, N_DEV):
            t = lax.rem(me + d, N_DEV)
            pltpu.make_async_remote_copy(
                src_ref=x2_ref.at[pl.ds(t * HALF, HALF), :],
                dst_ref=asm_ref.at[me],
                send_sem=send_sems.at[d],
                recv_sem=recv_sems.at[me],
                device_id=(t,),
                device_id_type=pl.DeviceIdType.MESH,
            ).wait_send()

    return pl.pallas_call(
        body,
        out_shape=jax.ShapeDtypeStruct((m_per, n), jnp.float32),
        in_specs=[
            pl.BlockSpec(memory_space=pltpu.VMEM),
            pl.BlockSpec(memory_space=pltpu.VMEM),
        ],
        out_specs=pl.BlockSpec(memory_space=pltpu.VMEM),
        scratch_shapes=[
            pltpu.VMEM((N_DEV, HALF, 2 * BLK), jnp.float32),
            pltpu.SemaphoreType.DMA((N_DEV,)),
            pltpu.SemaphoreType.DMA((N_DEV,)),
            pltpu.SemaphoreType.REGULAR((N_DEV,)),
        ],
        compiler_params=pltpu.CompilerParams(collective_id=0),
    )(x2, w_mat)
